# Initial kernel scaffold; baseline (speedup 1.0000x reference)
import jax
import jax.numpy as jnp
from jax.experimental import pallas as pl


def kernel(x, XY_Adj, params):
    raise NotImplementedError("write your pallas kernel here")


# speedup vs baseline: 1.0291x; 1.0291x over previous
import functools

import jax
import jax.numpy as jnp
from jax.experimental import pallas as pl

_N = 10000
_E = 40000


def _mm_body(x_ref, w_ref, b_ref, o_ref):
    o_ref[...] = (
        jnp.dot(x_ref[...], w_ref[...], preferred_element_type=jnp.float32)
        + b_ref[...]
    )


@functools.partial(jax.jit, static_argnames=("bm", "bn"))
def _fused_matmul(x, w, b, bm=1000, bn=1024):
    m, kdim = x.shape
    _, n = w.shape
    b2 = b.reshape(1, n)
    return pl.pallas_call(
        _mm_body,
        grid=(m // bm, n // bn),
        in_specs=[
            pl.BlockSpec((bm, kdim), lambda i, j: (i, 0)),
            pl.BlockSpec((kdim, bn), lambda i, j: (0, j)),
            pl.BlockSpec((1, bn), lambda i, j: (0, j)),
        ],
        out_specs=pl.BlockSpec((bm, bn), lambda i, j: (i, j)),
        out_shape=jax.ShapeDtypeStruct((m, n), jnp.float32),
    )(x, w, b2)


def _conv_layer(x, src, dst, valid, p):
    d = p["Wq"].shape[1]
    w4 = jnp.concatenate([p["Wq"], p["Wk"], p["Wv"], p["Ws"]], axis=1)
    b4 = jnp.concatenate([p["bq"], p["bk"], p["bv"], p["bs"]], axis=0)
    qkvs = _fused_matmul(x, w4, b4)
    q = qkvs[:, :d]
    k = qkvs[:, d : 2 * d]
    v = qkvs[:, 2 * d : 3 * d]
    s = qkvs[:, 3 * d :]
    alpha = jnp.sum(q[dst] * k[src], axis=-1) / jnp.sqrt(jnp.float32(d))
    ex = jnp.where(valid, jnp.exp(alpha), 0.0)
    den = jax.ops.segment_sum(ex, dst, num_segments=_N)
    agg = jax.ops.segment_sum(ex[:, None] * v[src], dst, num_segments=_N)
    return agg / (den[:, None] + 1e-16) + s


def kernel(x, XY_Adj, params):
    src, dst = jnp.nonzero(XY_Adj, size=_E, fill_value=0)
    cnt = jnp.sum(XY_Adj).astype(jnp.int32)
    valid = jnp.arange(_E) < cnt
    h1 = jax.nn.elu(_conv_layer(x, src, dst, valid, params["conv1"]))
    h2 = _conv_layer(h1, src, dst, valid, params["conv2"])
    h3 = jax.nn.elu(_conv_layer(h2, src, dst, valid, params["conv3"]))
    out = _conv_layer(h3, src, dst, valid, params["conv4"])
    return out
